# baseline (speedup 1.0000x reference)
import jax
import jax.numpy as jnp
from jax import lax
from jax.experimental import pallas as pl
from jax.experimental.pallas import tpu as pltpu
from jax.experimental.pallas import tpu_sc as plsc

N = 16384
C = 16
DIM = 64
SPAN = 256
NW = 32
SPW = N // NW
RPC = 128
SPC = RPC // C
NCHUNK = SPW // SPC
L = 16


def _sc_pool_body(candT_hbm, scoresT_hbm, table_hbm, tailT_hbm,
                  candT_v, scoresT_v, pair_v, off_v, rows0_v, rows1_v,
                  tailT_v, sem0, sem1):
    wid = lax.axis_index("s") * 2 + lax.axis_index("c")
    base = wid * SPW

    pltpu.sync_copy(candT_hbm.at[:, pl.ds(base, SPW)], candT_v)
    pltpu.sync_copy(scoresT_hbm.at[:, pl.ds(base, SPW)], scoresT_v)

    iota = lax.iota(jnp.int32, L)

    def build(j, _):
        for jj in range(SPC):
            s = j * SPC + jj
            cv = plsc.load_gather(candT_v, [iota, jnp.full((L,), s, jnp.int32)])
            pair_v[j, pl.ds(jj * C, L)] = (
                lax.shift_left(lax.shift_right_logical(cv, 11), 10)
                + (cv & 1023))
            off_v[j, pl.ds(jj * C, L)] = (
                lax.shift_right_logical(cv, 10) & 1) * DIM
        return ()

    lax.fori_loop(0, NCHUNK, build, (), unroll=False)

    def gather_start(j, rows, s):
        pltpu.async_copy(table_hbm.at[pair_v.at[j]], rows, s)

    def gather_wait(j, rows, s):
        pltpu.make_async_copy(table_hbm.at[pair_v.at[j]], rows, s).wait()

    def compute(j, rows):
        for jj in range(SPC):
            s = j * SPC + jj
            sidx = jnp.full((L,), s, jnp.int32)
            accs = [jnp.zeros((L,), jnp.float32) for _ in range(DIM // L)]
            offv = off_v[j, pl.ds(jj * C, C)]
            for c in range(C):
                r = jj * C + c
                bc = plsc.load_gather(
                    scoresT_v, [jnp.full((L,), c, jnp.int32), sidx])
                off = offv[c]
                for d in range(DIM // L):
                    accs[d] = accs[d] + bc * rows[r, pl.ds(off + d * L, L)]
            for d in range(DIM // L):
                plsc.store_scatter(
                    tailT_v, [d * L + iota, sidx], accs[d])

    gather_start(0, rows0_v, sem0)

    def step(j2, _):
        j = j2 * 2
        gather_wait(j, rows0_v, sem0)
        gather_start(j + 1, rows1_v, sem1)
        compute(j, rows0_v)
        gather_wait(j + 1, rows1_v, sem1)

        @pl.when(j2 + 1 < NCHUNK // 2)
        def _():
            gather_start(j + 2, rows0_v, sem0)

        compute(j + 1, rows1_v)
        return ()

    lax.fori_loop(0, NCHUNK // 2, step, (), unroll=False)

    pltpu.sync_copy(tailT_v, tailT_hbm.at[:, pl.ds(base, SPW)])


@jax.jit
def _sc_pool(candT, scoresT, table):
    mesh = plsc.VectorSubcoreMesh(core_axis_name="c", subcore_axis_name="s")
    f = pl.kernel(
        _sc_pool_body,
        out_type=jax.ShapeDtypeStruct((DIM, N), jnp.float32),
        mesh=mesh,
        compiler_params=pltpu.CompilerParams(needs_layout_passes=False),
        scratch_types=[
            pltpu.VMEM((C, SPW), jnp.int32),
            pltpu.VMEM((C, SPW), jnp.float32),
            pltpu.VMEM((NCHUNK, RPC), jnp.int32),
            pltpu.VMEM((NCHUNK, RPC), jnp.int32),
            pltpu.VMEM((RPC, 2 * DIM), jnp.float32),
            pltpu.VMEM((RPC, 2 * DIM), jnp.float32),
            pltpu.VMEM((DIM, SPW), jnp.float32),
            pltpu.SemaphoreType.DMA,
            pltpu.SemaphoreType.DMA,
        ],
    )
    return f(candT, scoresT, table)


NBLK = 489
UPB = 8
NUNIT = NBLK * UPB
V = 1000000


def _sc_relayout_body(embT_hbm, table_hbm, lo0, hi0, row0, lo1, hi1,
                      row1, seml0, semh0, seml1, semh1, semo0, semo1):
    wid = lax.axis_index("s") * 2 + lax.axis_index("c")
    iota = lax.iota(jnp.int32, L)

    def unit_start(t, lo_v, hi_v, sl, sh):
        u = wid + 32 * t
        g = u // UPB
        e_lo = g * 2048 + (u % UPB) * 128

        @pl.when(u < NUNIT)
        def _():
            @pl.when(e_lo + 128 <= V)
            def _():
                pltpu.async_copy(embT_hbm.at[:, pl.ds(e_lo, 128)], lo_v, sl)

            @pl.when(e_lo + 1024 + 128 <= V)
            def _():
                pltpu.async_copy(embT_hbm.at[:, pl.ds(e_lo + 1024, 128)],
                                 hi_v, sh)

    def unit_finish(t, lo_v, hi_v, row_v, sl, sh, so):
        u = wid + 32 * t
        g = u // UPB
        e_lo = g * 2048 + (u % UPB) * 128

        @pl.when(u < NUNIT)
        def _():
            @pl.when(e_lo + 128 <= V)
            def _():
                pltpu.make_async_copy(
                    embT_hbm.at[:, pl.ds(e_lo, 128)], lo_v, sl).wait()

            @pl.when(e_lo + 1024 + 128 <= V)
            def _():
                pltpu.make_async_copy(
                    embT_hbm.at[:, pl.ds(e_lo + 1024, 128)], hi_v, sh).wait()

            for c in range(128):
                cf = jnp.full((L,), c, jnp.int32)
                for d in range(DIM // L):
                    row_v[c, pl.ds(d * L, L)] = plsc.load_gather(
                        lo_v, [d * L + iota, cf])
                    row_v[c, pl.ds(DIM + d * L, L)] = plsc.load_gather(
                        hi_v, [d * L + iota, cf])
            pltpu.async_copy(
                row_v, table_hbm.at[pl.ds(g * 1024 + (u % UPB) * 128, 128)],
                so)

    def unit_drain(t, row_v, so):
        u = wid + 32 * t
        g = u // UPB

        @pl.when(u < NUNIT)
        def _():
            pltpu.make_async_copy(
                row_v, table_hbm.at[pl.ds(g * 1024 + (u % UPB) * 128, 128)],
                so).wait()

    nt = -(-NUNIT // 32)
    unit_start(0, lo0, hi0, seml0, semh0)

    def step(t2, _):
        t = t2 * 2
        unit_start(t + 1, lo1, hi1, seml1, semh1)
        unit_finish(t, lo0, hi0, row0, seml0, semh0, semo0)

        @pl.when(t + 2 < nt + (nt % 2))
        def _():
            unit_start(t + 2, lo0, hi0, seml0, semh0)

        unit_finish(t + 1, lo1, hi1, row1, seml1, semh1, semo1)
        unit_drain(t, row0, semo0)
        unit_drain(t + 1, row1, semo1)
        return ()

    lax.fori_loop(0, (nt + 1) // 2, step, (), unroll=False)


@jax.jit
def _sc_relayout(embT):
    mesh = plsc.VectorSubcoreMesh(core_axis_name="c", subcore_axis_name="s")
    f = pl.kernel(
        _sc_relayout_body,
        out_type=jax.ShapeDtypeStruct((NBLK * 1024, 2 * DIM), jnp.float32),
        mesh=mesh,
        compiler_params=pltpu.CompilerParams(needs_layout_passes=False),
        scratch_types=[
            pltpu.VMEM((DIM, 128), jnp.float32),
            pltpu.VMEM((DIM, 128), jnp.float32),
            pltpu.VMEM((128, 2 * DIM), jnp.float32),
            pltpu.VMEM((DIM, 128), jnp.float32),
            pltpu.VMEM((DIM, 128), jnp.float32),
            pltpu.VMEM((128, 2 * DIM), jnp.float32),
            pltpu.SemaphoreType.DMA,
            pltpu.SemaphoreType.DMA,
            pltpu.SemaphoreType.DMA,
            pltpu.SemaphoreType.DMA,
            pltpu.SemaphoreType.DMA,
            pltpu.SemaphoreType.DMA,
        ],
    )
    return f(embT)


def _assemble_body(span_ref, tail_ref, len_ref, out_ref):
    sv = span_ref[...]
    out_ref[:SPAN, :] = sv.T
    gate = (len_ref[...] > 0).astype(jnp.float32)
    out_ref[SPAN:, :] = tail_ref[...] * gate


@jax.jit
def _assemble(span_flat, tailT, len_row):
    blk = 512
    return pl.pallas_call(
        _assemble_body,
        grid=(N // blk,),
        in_specs=[
            pl.BlockSpec((blk, SPAN), lambda i: (i, 0)),
            pl.BlockSpec((DIM, blk), lambda i: (0, i)),
            pl.BlockSpec((1, blk), lambda i: (0, i)),
        ],
        out_specs=pl.BlockSpec((SPAN + DIM, blk), lambda i: (0, i)),
        out_shape=jax.ShapeDtypeStruct((SPAN + DIM, N), jnp.float32),
    )(span_flat, tailT, len_row)


def kernel(span_vecs, scores, mask_candidates, embed, candidates, len_candidates):
    candT = candidates[0].T
    scoresT = scores[0].T
    table = _sc_relayout(embed.T)
    table = jax.lax.dynamic_update_slice(table, embed[V - DIM:, :],
                                         (500224, 0))
    tailT = _sc_pool(candT, scoresT, table)
    outT = _assemble(span_vecs.reshape(N, SPAN), tailT,
                     len_candidates.reshape(1, N))
    return outT.T.reshape(1, N, SPAN + DIM)


# speedup vs baseline: 1.0012x; 1.0012x over previous
import jax
import jax.numpy as jnp
from jax import lax
from jax.experimental import pallas as pl
from jax.experimental.pallas import tpu as pltpu
from jax.experimental.pallas import tpu_sc as plsc

N = 16384
C = 16
DIM = 64
SPAN = 256
NW = 32
SPW = N // NW
RPC = 128
SPC = RPC // C
NCHUNK = SPW // SPC
L = 16


def _sc_pool_body(candT_hbm, scoresT_hbm, table_hbm, tailT_hbm,
                  candT_v, scoresT_v, pair_v, off_v, rows0_v, rows1_v,
                  tailT_v, sem0, sem1):
    wid = lax.axis_index("s") * 2 + lax.axis_index("c")
    base = wid * SPW

    pltpu.sync_copy(candT_hbm.at[:, pl.ds(base, SPW)], candT_v)
    pltpu.sync_copy(scoresT_hbm.at[:, pl.ds(base, SPW)], scoresT_v)

    iota = lax.iota(jnp.int32, L)

    def build(j, _):
        for jj in range(SPC):
            s = j * SPC + jj
            cv = plsc.load_gather(candT_v, [iota, jnp.full((L,), s, jnp.int32)])
            pair_v[j, pl.ds(jj * C, L)] = (
                lax.shift_left(lax.shift_right_logical(cv, 11), 10)
                + (cv & 1023))
            off_v[j, pl.ds(jj * C, L)] = (
                lax.shift_right_logical(cv, 10) & 1) * DIM
        return ()

    lax.fori_loop(0, NCHUNK, build, (), unroll=False)

    def gather_start(j, rows, s):
        pltpu.async_copy(table_hbm.at[pair_v.at[j]], rows, s)

    def gather_wait(j, rows, s):
        pltpu.make_async_copy(table_hbm.at[pair_v.at[j]], rows, s).wait()

    def compute(j, rows):
        for jj in range(SPC):
            s = j * SPC + jj
            sidx = jnp.full((L,), s, jnp.int32)
            accs = [jnp.zeros((L,), jnp.float32) for _ in range(DIM // L)]
            offv = off_v[j, pl.ds(jj * C, C)]
            for c in range(C):
                r = jj * C + c
                bc = plsc.load_gather(
                    scoresT_v, [jnp.full((L,), c, jnp.int32), sidx])
                off = offv[c]
                for d in range(DIM // L):
                    accs[d] = accs[d] + bc * rows[r, pl.ds(off + d * L, L)]
            for d in range(DIM // L):
                plsc.store_scatter(
                    tailT_v, [d * L + iota, sidx], accs[d])

    gather_start(0, rows0_v, sem0)

    def step(j2, _):
        j = j2 * 2
        gather_wait(j, rows0_v, sem0)
        gather_start(j + 1, rows1_v, sem1)
        compute(j, rows0_v)
        gather_wait(j + 1, rows1_v, sem1)

        @pl.when(j2 + 1 < NCHUNK // 2)
        def _():
            gather_start(j + 2, rows0_v, sem0)

        compute(j + 1, rows1_v)
        return ()

    lax.fori_loop(0, NCHUNK // 2, step, (), unroll=False)

    pltpu.sync_copy(tailT_v, tailT_hbm.at[:, pl.ds(base, SPW)])


@jax.jit
def _sc_pool(candT, scoresT, table):
    mesh = plsc.VectorSubcoreMesh(core_axis_name="c", subcore_axis_name="s")
    f = pl.kernel(
        _sc_pool_body,
        out_type=jax.ShapeDtypeStruct((DIM, N), jnp.float32),
        mesh=mesh,
        compiler_params=pltpu.CompilerParams(needs_layout_passes=False),
        scratch_types=[
            pltpu.VMEM((C, SPW), jnp.int32),
            pltpu.VMEM((C, SPW), jnp.float32),
            pltpu.VMEM((NCHUNK, RPC), jnp.int32),
            pltpu.VMEM((NCHUNK, RPC), jnp.int32),
            pltpu.VMEM((RPC, 2 * DIM), jnp.float32),
            pltpu.VMEM((RPC, 2 * DIM), jnp.float32),
            pltpu.VMEM((DIM, SPW), jnp.float32),
            pltpu.SemaphoreType.DMA,
            pltpu.SemaphoreType.DMA,
        ],
    )
    return f(candT, scoresT, table)


NBLK = 489
UPB = 8
NUNIT = NBLK * UPB
V = 1000000


def _sc_relayout_body(embT_hbm, table_hbm, lo0, hi0, row0, lo1, hi1,
                      row1, seml0, semh0, seml1, semh1, semo0, semo1):
    wid = lax.axis_index("s") * 2 + lax.axis_index("c")
    iota = lax.iota(jnp.int32, L)

    def unit_start(t, lo_v, hi_v, sl, sh):
        u = wid + 32 * t
        g = u // UPB
        e_lo = g * 2048 + (u % UPB) * 128

        @pl.when(u < NUNIT)
        def _():
            @pl.when(e_lo + 128 <= V)
            def _():
                pltpu.async_copy(embT_hbm.at[:, pl.ds(e_lo, 128)], lo_v, sl)

            @pl.when(e_lo + 1024 + 128 <= V)
            def _():
                pltpu.async_copy(embT_hbm.at[:, pl.ds(e_lo + 1024, 128)],
                                 hi_v, sh)

    def unit_finish(t, lo_v, hi_v, row_v, sl, sh, so):
        u = wid + 32 * t
        g = u // UPB
        e_lo = g * 2048 + (u % UPB) * 128

        @pl.when(u < NUNIT)
        def _():
            @pl.when(e_lo + 128 <= V)
            def _():
                pltpu.make_async_copy(
                    embT_hbm.at[:, pl.ds(e_lo, 128)], lo_v, sl).wait()

            @pl.when(e_lo + 1024 + 128 <= V)
            def _():
                pltpu.make_async_copy(
                    embT_hbm.at[:, pl.ds(e_lo + 1024, 128)], hi_v, sh).wait()

            def cols(c16, _):
                for cc in range(16):
                    c = c16 * 16 + cc
                    cf = jnp.full((L,), c, jnp.int32)
                    for d in range(DIM // L):
                        row_v[c, pl.ds(d * L, L)] = plsc.load_gather(
                            lo_v, [d * L + iota, cf])
                        row_v[c, pl.ds(DIM + d * L, L)] = plsc.load_gather(
                            hi_v, [d * L + iota, cf])
                return ()

            lax.fori_loop(0, 8, cols, (), unroll=False)
            pltpu.async_copy(
                row_v, table_hbm.at[pl.ds(g * 1024 + (u % UPB) * 128, 128)],
                so)

    def unit_drain(t, row_v, so):
        u = wid + 32 * t
        g = u // UPB

        @pl.when(u < NUNIT)
        def _():
            pltpu.make_async_copy(
                row_v, table_hbm.at[pl.ds(g * 1024 + (u % UPB) * 128, 128)],
                so).wait()

    nt = -(-NUNIT // 32)
    unit_start(0, lo0, hi0, seml0, semh0)

    def step(t2, _):
        t = t2 * 2
        unit_start(t + 1, lo1, hi1, seml1, semh1)
        unit_finish(t, lo0, hi0, row0, seml0, semh0, semo0)

        @pl.when(t + 2 < nt + (nt % 2))
        def _():
            unit_start(t + 2, lo0, hi0, seml0, semh0)

        unit_finish(t + 1, lo1, hi1, row1, seml1, semh1, semo1)
        unit_drain(t, row0, semo0)
        unit_drain(t + 1, row1, semo1)
        return ()

    lax.fori_loop(0, (nt + 1) // 2, step, (), unroll=False)


@jax.jit
def _sc_relayout(embT):
    mesh = plsc.VectorSubcoreMesh(core_axis_name="c", subcore_axis_name="s")
    f = pl.kernel(
        _sc_relayout_body,
        out_type=jax.ShapeDtypeStruct((NBLK * 1024, 2 * DIM), jnp.float32),
        mesh=mesh,
        compiler_params=pltpu.CompilerParams(needs_layout_passes=False),
        scratch_types=[
            pltpu.VMEM((DIM, 128), jnp.float32),
            pltpu.VMEM((DIM, 128), jnp.float32),
            pltpu.VMEM((128, 2 * DIM), jnp.float32),
            pltpu.VMEM((DIM, 128), jnp.float32),
            pltpu.VMEM((DIM, 128), jnp.float32),
            pltpu.VMEM((128, 2 * DIM), jnp.float32),
            pltpu.SemaphoreType.DMA,
            pltpu.SemaphoreType.DMA,
            pltpu.SemaphoreType.DMA,
            pltpu.SemaphoreType.DMA,
            pltpu.SemaphoreType.DMA,
            pltpu.SemaphoreType.DMA,
        ],
    )
    return f(embT)


def _assemble_body(span_ref, tail_ref, len_ref, out_ref):
    sv = span_ref[...]
    out_ref[:SPAN, :] = sv.T
    gate = (len_ref[...] > 0).astype(jnp.float32)
    out_ref[SPAN:, :] = tail_ref[...] * gate


@jax.jit
def _assemble(span_flat, tailT, len_row):
    blk = 512
    return pl.pallas_call(
        _assemble_body,
        grid=(N // blk,),
        in_specs=[
            pl.BlockSpec((blk, SPAN), lambda i: (i, 0)),
            pl.BlockSpec((DIM, blk), lambda i: (0, i)),
            pl.BlockSpec((1, blk), lambda i: (0, i)),
        ],
        out_specs=pl.BlockSpec((SPAN + DIM, blk), lambda i: (0, i)),
        out_shape=jax.ShapeDtypeStruct((SPAN + DIM, N), jnp.float32),
    )(span_flat, tailT, len_row)


def kernel(span_vecs, scores, mask_candidates, embed, candidates, len_candidates):
    candT = candidates[0].T
    scoresT = scores[0].T
    table = _sc_relayout(embed.T)
    table = jax.lax.dynamic_update_slice(table, embed[V - DIM:, :],
                                         (500224, 0))
    tailT = _sc_pool(candT, scoresT, table)
    outT = _assemble(span_vecs.reshape(N, SPAN), tailT,
                     len_candidates.reshape(1, N))
    return outT.T.reshape(1, N, SPAN + DIM)


# speedup vs baseline: 1.0014x; 1.0002x over previous
import jax
import jax.numpy as jnp
from jax import lax
from jax.experimental import pallas as pl
from jax.experimental.pallas import tpu as pltpu
from jax.experimental.pallas import tpu_sc as plsc

N = 16384
C = 16
DIM = 64
SPAN = 256
NW = 32
SPW = N // NW
RPC = 128
SPC = RPC // C
NCHUNK = SPW // SPC
L = 16


def _sc_pool_body(candT_hbm, scoresT_hbm, table_hbm, tailT_hbm,
                  candT_v, scoresT_v, pair_v, off_v, rows0_v, rows1_v,
                  tailT_v, sem0, sem1):
    wid = lax.axis_index("s") * 2 + lax.axis_index("c")
    base = wid * SPW

    pltpu.sync_copy(candT_hbm.at[:, pl.ds(base, SPW)], candT_v)
    pltpu.sync_copy(scoresT_hbm.at[:, pl.ds(base, SPW)], scoresT_v)

    iota = lax.iota(jnp.int32, L)

    def build(j, _):
        for jj in range(SPC):
            s = j * SPC + jj
            cv = plsc.load_gather(candT_v, [iota, jnp.full((L,), s, jnp.int32)])
            pair_v[j, pl.ds(jj * C, L)] = (
                lax.shift_left(lax.shift_right_logical(cv, 11), 10)
                + (cv & 1023))
            off_v[j, pl.ds(jj * C, L)] = (
                lax.shift_right_logical(cv, 10) & 1) * DIM
        return ()

    lax.fori_loop(0, NCHUNK, build, (), unroll=False)

    def gather_start(j, rows, s):
        pltpu.async_copy(table_hbm.at[pair_v.at[j]], rows, s)

    def gather_wait(j, rows, s):
        pltpu.make_async_copy(table_hbm.at[pair_v.at[j]], rows, s).wait()

    def compute(j, rows):
        for jj in range(SPC):
            s = j * SPC + jj
            sidx = jnp.full((L,), s, jnp.int32)
            accs = [jnp.zeros((L,), jnp.float32) for _ in range(DIM // L)]
            offv = off_v[j, pl.ds(jj * C, C)]
            for c in range(C):
                r = jj * C + c
                bc = plsc.load_gather(
                    scoresT_v, [jnp.full((L,), c, jnp.int32), sidx])
                off = offv[c]
                for d in range(DIM // L):
                    accs[d] = accs[d] + bc * rows[r, pl.ds(off + d * L, L)]
            for d in range(DIM // L):
                plsc.store_scatter(
                    tailT_v, [d * L + iota, sidx], accs[d])

    gather_start(0, rows0_v, sem0)

    def step(j2, _):
        j = j2 * 2
        gather_wait(j, rows0_v, sem0)
        gather_start(j + 1, rows1_v, sem1)
        compute(j, rows0_v)
        gather_wait(j + 1, rows1_v, sem1)

        @pl.when(j2 + 1 < NCHUNK // 2)
        def _():
            gather_start(j + 2, rows0_v, sem0)

        compute(j + 1, rows1_v)
        return ()

    lax.fori_loop(0, NCHUNK // 2, step, (), unroll=False)

    pltpu.sync_copy(tailT_v, tailT_hbm.at[:, pl.ds(base, SPW)])


@jax.jit
def _sc_pool(candT, scoresT, table):
    mesh = plsc.VectorSubcoreMesh(core_axis_name="c", subcore_axis_name="s")
    f = pl.kernel(
        _sc_pool_body,
        out_type=jax.ShapeDtypeStruct((DIM, N), jnp.float32),
        mesh=mesh,
        compiler_params=pltpu.CompilerParams(needs_layout_passes=False),
        scratch_types=[
            pltpu.VMEM((C, SPW), jnp.int32),
            pltpu.VMEM((C, SPW), jnp.float32),
            pltpu.VMEM((NCHUNK, RPC), jnp.int32),
            pltpu.VMEM((NCHUNK, RPC), jnp.int32),
            pltpu.VMEM((RPC, 2 * DIM), jnp.float32),
            pltpu.VMEM((RPC, 2 * DIM), jnp.float32),
            pltpu.VMEM((DIM, SPW), jnp.float32),
            pltpu.SemaphoreType.DMA,
            pltpu.SemaphoreType.DMA,
        ],
    )
    return f(candT, scoresT, table)


NBLK = 489
UPB = 8
NUNIT = NBLK * UPB
V = 1000000


def _sc_relayout_body(embT_hbm, table_hbm, lo0, hi0, row0, lo1, hi1,
                      row1, seml0, semh0, seml1, semh1, semo0, semo1):
    wid = lax.axis_index("s") * 2 + lax.axis_index("c")
    iota = lax.iota(jnp.int32, L)

    def unit_start(t, lo_v, hi_v, sl, sh):
        u = wid + 32 * t
        g = u // UPB
        e_lo = g * 2048 + (u % UPB) * 128

        @pl.when(u < NUNIT)
        def _():
            @pl.when(e_lo + 128 <= V)
            def _():
                pltpu.async_copy(embT_hbm.at[:, pl.ds(e_lo, 128)],
                                 lo_v.at[:, pl.ds(0, 128)], sl)

            @pl.when(e_lo + 1024 + 128 <= V)
            def _():
                pltpu.async_copy(embT_hbm.at[:, pl.ds(e_lo + 1024, 128)],
                                 hi_v.at[:, pl.ds(0, 128)], sh)

    def unit_finish(t, lo_v, hi_v, row_v, sl, sh, so):
        u = wid + 32 * t
        g = u // UPB
        e_lo = g * 2048 + (u % UPB) * 128

        @pl.when(u < NUNIT)
        def _():
            @pl.when(e_lo + 128 <= V)
            def _():
                pltpu.make_async_copy(
                    embT_hbm.at[:, pl.ds(e_lo, 128)],
                    lo_v.at[:, pl.ds(0, 128)], sl).wait()

            @pl.when(e_lo + 1024 + 128 <= V)
            def _():
                pltpu.make_async_copy(
                    embT_hbm.at[:, pl.ds(e_lo + 1024, 128)],
                    hi_v.at[:, pl.ds(0, 128)], sh).wait()

            def cols(c16, _):
                for cc in range(16):
                    c = c16 * 16 + cc
                    cf = jnp.full((L,), c, jnp.int32)
                    for d in range(DIM // L):
                        row_v[c, pl.ds(d * L, L)] = plsc.load_gather(
                            lo_v, [d * L + iota, cf])
                        row_v[c, pl.ds(DIM + d * L, L)] = plsc.load_gather(
                            hi_v, [d * L + iota, cf])
                return ()

            lax.fori_loop(0, 8, cols, (), unroll=False)
            pltpu.async_copy(
                row_v, table_hbm.at[pl.ds(g * 1024 + (u % UPB) * 128, 128)],
                so)

    def unit_drain(t, row_v, so):
        u = wid + 32 * t
        g = u // UPB

        @pl.when(u < NUNIT)
        def _():
            pltpu.make_async_copy(
                row_v, table_hbm.at[pl.ds(g * 1024 + (u % UPB) * 128, 128)],
                so).wait()

    nt = -(-NUNIT // 32)
    unit_start(0, lo0, hi0, seml0, semh0)

    def step(t2, _):
        t = t2 * 2
        unit_start(t + 1, lo1, hi1, seml1, semh1)
        unit_finish(t, lo0, hi0, row0, seml0, semh0, semo0)

        @pl.when(t + 2 < nt + (nt % 2))
        def _():
            unit_start(t + 2, lo0, hi0, seml0, semh0)

        unit_finish(t + 1, lo1, hi1, row1, seml1, semh1, semo1)
        unit_drain(t, row0, semo0)
        unit_drain(t + 1, row1, semo1)
        return ()

    lax.fori_loop(0, (nt + 1) // 2, step, (), unroll=False)


@jax.jit
def _sc_relayout(embT):
    mesh = plsc.VectorSubcoreMesh(core_axis_name="c", subcore_axis_name="s")
    f = pl.kernel(
        _sc_relayout_body,
        out_type=jax.ShapeDtypeStruct((NBLK * 1024, 2 * DIM), jnp.float32),
        mesh=mesh,
        compiler_params=pltpu.CompilerParams(needs_layout_passes=False),
        scratch_types=[
            pltpu.VMEM((DIM, 129), jnp.float32),
            pltpu.VMEM((DIM, 129), jnp.float32),
            pltpu.VMEM((128, 2 * DIM), jnp.float32),
            pltpu.VMEM((DIM, 129), jnp.float32),
            pltpu.VMEM((DIM, 129), jnp.float32),
            pltpu.VMEM((128, 2 * DIM), jnp.float32),
            pltpu.SemaphoreType.DMA,
            pltpu.SemaphoreType.DMA,
            pltpu.SemaphoreType.DMA,
            pltpu.SemaphoreType.DMA,
            pltpu.SemaphoreType.DMA,
            pltpu.SemaphoreType.DMA,
        ],
    )
    return f(embT)


def _assemble_body(span_ref, tail_ref, len_ref, out_ref):
    sv = span_ref[...]
    out_ref[:SPAN, :] = sv.T
    gate = (len_ref[...] > 0).astype(jnp.float32)
    out_ref[SPAN:, :] = tail_ref[...] * gate


@jax.jit
def _assemble(span_flat, tailT, len_row):
    blk = 512
    return pl.pallas_call(
        _assemble_body,
        grid=(N // blk,),
        in_specs=[
            pl.BlockSpec((blk, SPAN), lambda i: (i, 0)),
            pl.BlockSpec((DIM, blk), lambda i: (0, i)),
            pl.BlockSpec((1, blk), lambda i: (0, i)),
        ],
        out_specs=pl.BlockSpec((SPAN + DIM, blk), lambda i: (0, i)),
        out_shape=jax.ShapeDtypeStruct((SPAN + DIM, N), jnp.float32),
    )(span_flat, tailT, len_row)


def kernel(span_vecs, scores, mask_candidates, embed, candidates, len_candidates):
    candT = candidates[0].T
    scoresT = scores[0].T
    table = _sc_relayout(embed.T)
    table = jax.lax.dynamic_update_slice(table, embed[V - DIM:, :],
                                         (500224, 0))
    tailT = _sc_pool(candT, scoresT, table)
    outT = _assemble(span_vecs.reshape(N, SPAN), tailT,
                     len_candidates.reshape(1, N))
    return outT.T.reshape(1, N, SPAN + DIM)


# speedup vs baseline: 2.9254x; 2.9213x over previous
import jax
import jax.numpy as jnp
from jax import lax
from jax.experimental import pallas as pl
from jax.experimental.pallas import tpu as pltpu
from jax.experimental.pallas import tpu_sc as plsc

N = 16384
C = 16
DIM = 64
SPAN = 256
NW = 32
SPW = N // NW
RPC = 128
SPC = RPC // C
NCHUNK = SPW // SPC
L = 16


def _sc_pool_body(candT_hbm, scoresT_hbm, table_hbm, tailT_hbm,
                  candT_v, scoresT_v, pair_v, off_v, rows0_v, rows1_v,
                  tailT_v, sem0, sem1):
    wid = lax.axis_index("s") * 2 + lax.axis_index("c")
    base = wid * SPW

    pltpu.sync_copy(candT_hbm.at[:, pl.ds(base, SPW)], candT_v)
    pltpu.sync_copy(scoresT_hbm.at[:, pl.ds(base, SPW)], scoresT_v)

    iota = lax.iota(jnp.int32, L)

    def build(j, _):
        for jj in range(SPC):
            s = j * SPC + jj
            cv = plsc.load_gather(candT_v, [iota, jnp.full((L,), s, jnp.int32)])
            pair_v[j, pl.ds(jj * C, L)] = lax.shift_right_logical(cv, 1)
            off_v[j, pl.ds(jj * C, L)] = (cv & 1) * DIM
        return ()

    lax.fori_loop(0, NCHUNK, build, (), unroll=False)

    def gather_start(j, rows, s):
        pltpu.async_copy(table_hbm.at[pair_v.at[j]], rows, s)

    def gather_wait(j, rows, s):
        pltpu.make_async_copy(table_hbm.at[pair_v.at[j]], rows, s).wait()

    def compute(j, rows):
        for jj in range(SPC):
            s = j * SPC + jj
            sidx = jnp.full((L,), s, jnp.int32)
            accs = [jnp.zeros((L,), jnp.float32) for _ in range(DIM // L)]
            offv = off_v[j, pl.ds(jj * C, C)]
            for c in range(C):
                r = jj * C + c
                bc = plsc.load_gather(
                    scoresT_v, [jnp.full((L,), c, jnp.int32), sidx])
                off = offv[c]
                for d in range(DIM // L):
                    accs[d] = accs[d] + bc * rows[r, pl.ds(off + d * L, L)]
            for d in range(DIM // L):
                plsc.store_scatter(
                    tailT_v, [d * L + iota, sidx], accs[d])

    gather_start(0, rows0_v, sem0)

    def step(j2, _):
        j = j2 * 2
        gather_wait(j, rows0_v, sem0)
        gather_start(j + 1, rows1_v, sem1)
        compute(j, rows0_v)
        gather_wait(j + 1, rows1_v, sem1)

        @pl.when(j2 + 1 < NCHUNK // 2)
        def _():
            gather_start(j + 2, rows0_v, sem0)

        compute(j + 1, rows1_v)
        return ()

    lax.fori_loop(0, NCHUNK // 2, step, (), unroll=False)

    pltpu.sync_copy(tailT_v, tailT_hbm.at[:, pl.ds(base, SPW)])


@jax.jit
def _sc_pool(candT, scoresT, table):
    mesh = plsc.VectorSubcoreMesh(core_axis_name="c", subcore_axis_name="s")
    f = pl.kernel(
        _sc_pool_body,
        out_type=jax.ShapeDtypeStruct((DIM, N), jnp.float32),
        mesh=mesh,
        compiler_params=pltpu.CompilerParams(needs_layout_passes=False),
        scratch_types=[
            pltpu.VMEM((C, SPW), jnp.int32),
            pltpu.VMEM((C, SPW), jnp.float32),
            pltpu.VMEM((NCHUNK, RPC), jnp.int32),
            pltpu.VMEM((NCHUNK, RPC), jnp.int32),
            pltpu.VMEM((RPC, 2 * DIM), jnp.float32),
            pltpu.VMEM((RPC, 2 * DIM), jnp.float32),
            pltpu.VMEM((DIM, SPW), jnp.float32),
            pltpu.SemaphoreType.DMA,
            pltpu.SemaphoreType.DMA,
        ],
    )
    return f(candT, scoresT, table)


def _relayout_body(embT_ref, p_ref, out_ref):
    p = p_ref[...]
    nstrip = embT_ref.shape[1] // 128
    for k4 in range(nstrip // 4):
        s4 = jnp.concatenate(
            [embT_ref[:, pl.ds((k4 * 4 + a) * 128, 128)] for a in range(4)],
            axis=0)
        mt = jax.lax.dot_general(p, s4, (((0,), (1,)), ((), ())),
                                 preferred_element_type=jnp.float32)
        for a in range(4):
            k = k4 * 4 + a
            out_ref[pl.ds(k * DIM, DIM), :DIM] = mt[:DIM, a * DIM:(a + 1) * DIM]
            out_ref[pl.ds(k * DIM, DIM), DIM:] = mt[DIM:, a * DIM:(a + 1) * DIM]


@jax.jit
def _tc_relayout(embT, p):
    b = 1024
    nblk = -(-embT.shape[1] // (2 * b))
    return pl.pallas_call(
        _relayout_body,
        grid=(nblk,),
        in_specs=[
            pl.BlockSpec((DIM, 2 * b), lambda i: (0, i)),
            pl.BlockSpec((128, 128), lambda i: (0, 0)),
        ],
        out_specs=pl.BlockSpec((b, 2 * DIM), lambda i: (i, 0)),
        out_shape=jax.ShapeDtypeStruct((embT.shape[1] // 2, 2 * DIM),
                                       jnp.float32),
    )(embT, p)


def _assemble_body(span_ref, tail_ref, len_ref, out_ref):
    sv = span_ref[...]
    out_ref[:SPAN, :] = sv.T
    gate = (len_ref[...] > 0).astype(jnp.float32)
    out_ref[SPAN:, :] = tail_ref[...] * gate


@jax.jit
def _assemble(span_flat, tailT, len_row):
    blk = 512
    return pl.pallas_call(
        _assemble_body,
        grid=(N // blk,),
        in_specs=[
            pl.BlockSpec((blk, SPAN), lambda i: (i, 0)),
            pl.BlockSpec((DIM, blk), lambda i: (0, i)),
            pl.BlockSpec((1, blk), lambda i: (0, i)),
        ],
        out_specs=pl.BlockSpec((SPAN + DIM, blk), lambda i: (0, i)),
        out_shape=jax.ShapeDtypeStruct((SPAN + DIM, N), jnp.float32),
    )(span_flat, tailT, len_row)


def kernel(span_vecs, scores, mask_candidates, embed, candidates, len_candidates):
    candT = candidates[0].T
    scoresT = scores[0].T
    j = jnp.arange(128)
    perm = jnp.where(j % 2 == 0, j // 2, DIM + j // 2)
    p = (jnp.arange(128)[None, :] == perm[:, None]).astype(jnp.float32)
    table = _tc_relayout(embed.T, p)
    tailT = _sc_pool(candT, scoresT, table)
    outT = _assemble(span_vecs.reshape(N, SPAN), tailT,
                     len_candidates.reshape(1, N))
    return outT.T.reshape(1, N, SPAN + DIM)


# speedup vs baseline: 3.8546x; 1.3176x over previous
import jax
import jax.numpy as jnp
from jax import lax
from jax.experimental import pallas as pl
from jax.experimental.pallas import tpu as pltpu
from jax.experimental.pallas import tpu_sc as plsc

N = 16384
C = 16
DIM = 64
SPAN = 256
NW = 32
SPW = N // NW
RPC = 128
SPC = RPC // C
NCHUNK = SPW // SPC
L = 16


def _sc_pool_body(candT_hbm, scoresT_hbm, table_hbm, tailT_hbm,
                  candT_v, scoresT_v, pair_v, off_v, rows0_v, rows1_v,
                  tailT_v, sem0, sem1):
    wid = lax.axis_index("s") * 2 + lax.axis_index("c")
    base = wid * SPW

    pltpu.sync_copy(candT_hbm.at[:, pl.ds(base, SPW)], candT_v)
    pltpu.sync_copy(scoresT_hbm.at[:, pl.ds(base, SPW)], scoresT_v)

    iota = lax.iota(jnp.int32, L)

    def build(j, _):
        for jj in range(SPC):
            s = j * SPC + jj
            cv = plsc.load_gather(candT_v, [iota, jnp.full((L,), s, jnp.int32)])
            pair_v[j, pl.ds(jj * C, L)] = lax.shift_right_logical(cv, 1)
            off_v[j, pl.ds(jj * C, L)] = (cv & 1) * DIM
        return ()

    lax.fori_loop(0, NCHUNK, build, (), unroll=False)

    def gather_start(j, rows, s):
        pltpu.async_copy(table_hbm.at[pair_v.at[j]], rows, s)

    def gather_wait(j, rows, s):
        pltpu.make_async_copy(table_hbm.at[pair_v.at[j]], rows, s).wait()

    def compute(j, rows):
        for jj in range(SPC):
            s = j * SPC + jj
            sidx = jnp.full((L,), s, jnp.int32)
            accs = [jnp.zeros((L,), jnp.float32) for _ in range(DIM // L)]
            offv = off_v[j, pl.ds(jj * C, C)]
            for c in range(C):
                r = jj * C + c
                bc = plsc.load_gather(
                    scoresT_v, [jnp.full((L,), c, jnp.int32), sidx])
                off = offv[c]
                for d in range(DIM // L):
                    accs[d] = accs[d] + bc * rows[r, pl.ds(off + d * L, L)]
            for d in range(DIM // L):
                plsc.store_scatter(
                    tailT_v, [d * L + iota, sidx], accs[d])

    gather_start(0, rows0_v, sem0)

    def step(j2, _):
        j = j2 * 2
        gather_wait(j, rows0_v, sem0)
        gather_start(j + 1, rows1_v, sem1)
        compute(j, rows0_v)
        gather_wait(j + 1, rows1_v, sem1)

        @pl.when(j2 + 1 < NCHUNK // 2)
        def _():
            gather_start(j + 2, rows0_v, sem0)

        compute(j + 1, rows1_v)
        return ()

    lax.fori_loop(0, NCHUNK // 2, step, (), unroll=False)

    pltpu.sync_copy(tailT_v, tailT_hbm.at[:, pl.ds(base, SPW)])


@jax.jit
def _sc_pool(candT, scoresT, table):
    mesh = plsc.VectorSubcoreMesh(core_axis_name="c", subcore_axis_name="s")
    f = pl.kernel(
        _sc_pool_body,
        out_type=jax.ShapeDtypeStruct((DIM, N), jnp.float32),
        mesh=mesh,
        compiler_params=pltpu.CompilerParams(needs_layout_passes=False),
        scratch_types=[
            pltpu.VMEM((C, SPW), jnp.int32),
            pltpu.VMEM((C, SPW), jnp.float32),
            pltpu.VMEM((NCHUNK, RPC), jnp.int32),
            pltpu.VMEM((NCHUNK, RPC), jnp.int32),
            pltpu.VMEM((RPC, 2 * DIM), jnp.float32),
            pltpu.VMEM((RPC, 2 * DIM), jnp.float32),
            pltpu.VMEM((DIM, SPW), jnp.float32),
            pltpu.SemaphoreType.DMA,
            pltpu.SemaphoreType.DMA,
        ],
    )
    return f(candT, scoresT, table)


def _relayout_body(embT_ref, p_ref, out_ref):
    p = p_ref[...]
    nstrip = embT_ref.shape[1] // 128
    for k4 in range(nstrip // 4):
        s4 = jnp.concatenate(
            [embT_ref[:, pl.ds((k4 * 4 + a) * 128, 128)] for a in range(4)],
            axis=0)
        mt = jax.lax.dot_general(p, s4, (((0,), (1,)), ((), ())),
                                 preferred_element_type=jnp.float32)
        for a in range(4):
            k = k4 * 4 + a
            out_ref[pl.ds(k * DIM, DIM), :DIM] = mt[:DIM, a * DIM:(a + 1) * DIM]
            out_ref[pl.ds(k * DIM, DIM), DIM:] = mt[DIM:, a * DIM:(a + 1) * DIM]


@jax.jit
def _tc_relayout(embT, p):
    b = 2048
    nblk = -(-embT.shape[1] // (2 * b))
    return pl.pallas_call(
        _relayout_body,
        grid=(nblk,),
        in_specs=[
            pl.BlockSpec((DIM, 2 * b), lambda i: (0, i)),
            pl.BlockSpec((128, 128), lambda i: (0, 0)),
        ],
        out_specs=pl.BlockSpec((b, 2 * DIM), lambda i: (i, 0)),
        out_shape=jax.ShapeDtypeStruct((embT.shape[1] // 2, 2 * DIM),
                                       jnp.float32),
    )(embT, p)


def _assemble_body(span_ref, tail_ref, len_ref, out_ref):
    sv = span_ref[...]
    out_ref[:SPAN, :] = sv.T
    gate = (len_ref[...] > 0).astype(jnp.float32)
    out_ref[SPAN:, :] = tail_ref[...] * gate


@jax.jit
def _assemble(span_flat, tailT, len_row):
    blk = 512
    return pl.pallas_call(
        _assemble_body,
        grid=(N // blk,),
        in_specs=[
            pl.BlockSpec((blk, SPAN), lambda i: (i, 0)),
            pl.BlockSpec((DIM, blk), lambda i: (0, i)),
            pl.BlockSpec((1, blk), lambda i: (0, i)),
        ],
        out_specs=pl.BlockSpec((SPAN + DIM, blk), lambda i: (0, i)),
        out_shape=jax.ShapeDtypeStruct((SPAN + DIM, N), jnp.float32),
    )(span_flat, tailT, len_row)


def kernel(span_vecs, scores, mask_candidates, embed, candidates, len_candidates):
    candT = candidates[0].T
    scoresT = scores[0].T
    j = jnp.arange(128)
    perm = jnp.where(j % 2 == 0, j // 2, DIM + j // 2)
    p = (jnp.arange(128)[None, :] == perm[:, None]).astype(jnp.float32)
    table = _tc_relayout(embed.T, p)
    tailT = _sc_pool(candT, scoresT, table)
    outT = _assemble(span_vecs.reshape(N, SPAN), tailT,
                     len_candidates.reshape(1, N))
    return outT.T.reshape(1, N, SPAN + DIM)


# speedup vs baseline: 4.7159x; 1.2234x over previous
import jax
import jax.numpy as jnp
from jax import lax
from jax.experimental import pallas as pl
from jax.experimental.pallas import tpu as pltpu
from jax.experimental.pallas import tpu_sc as plsc

N = 16384
C = 16
DIM = 64
SPAN = 256
NW = 32
SPW = N // NW
RPC = 128
SPC = RPC // C
NCHUNK = SPW // SPC
L = 16


def _sc_pool_body(candT_hbm, scoresT_hbm, table_hbm, tailT_hbm,
                  candT_v, scoresT_v, pair_v, off_v, rows0_v, rows1_v,
                  tailT_v, sem0, sem1):
    wid = lax.axis_index("s") * 2 + lax.axis_index("c")
    base = wid * SPW

    pltpu.sync_copy(candT_hbm.at[:, pl.ds(base, SPW)], candT_v)
    pltpu.sync_copy(scoresT_hbm.at[:, pl.ds(base, SPW)], scoresT_v)

    iota = lax.iota(jnp.int32, L)

    def build(j, _):
        for jj in range(SPC):
            s = j * SPC + jj
            cv = plsc.load_gather(candT_v, [iota, jnp.full((L,), s, jnp.int32)])
            pair_v[j, pl.ds(jj * C, L)] = lax.shift_right_logical(cv, 1)
            off_v[j, pl.ds(jj * C, L)] = (cv & 1) * DIM
        return ()

    lax.fori_loop(0, NCHUNK, build, (), unroll=False)

    def gather_start(j, rows, s):
        pltpu.async_copy(table_hbm.at[pair_v.at[j]], rows, s)

    def gather_wait(j, rows, s):
        pltpu.make_async_copy(table_hbm.at[pair_v.at[j]], rows, s).wait()

    def compute(j, rows):
        for jj in range(SPC):
            s = j * SPC + jj
            sidx = jnp.full((L,), s, jnp.int32)
            accs = [jnp.zeros((L,), jnp.float32) for _ in range(DIM // L)]
            offv = off_v[j, pl.ds(jj * C, C)]
            for c in range(C):
                r = jj * C + c
                bc = plsc.load_gather(
                    scoresT_v, [jnp.full((L,), c, jnp.int32), sidx])
                off = offv[c]
                for d in range(DIM // L):
                    accs[d] = accs[d] + bc * rows[r, pl.ds(off + d * L, L)]
            for d in range(DIM // L):
                plsc.store_scatter(
                    tailT_v, [d * L + iota, sidx], accs[d])

    gather_start(0, rows0_v, sem0)

    def step(j2, _):
        j = j2 * 2
        gather_wait(j, rows0_v, sem0)
        gather_start(j + 1, rows1_v, sem1)
        compute(j, rows0_v)
        gather_wait(j + 1, rows1_v, sem1)

        @pl.when(j2 + 1 < NCHUNK // 2)
        def _():
            gather_start(j + 2, rows0_v, sem0)

        compute(j + 1, rows1_v)
        return ()

    lax.fori_loop(0, NCHUNK // 2, step, (), unroll=False)

    pltpu.sync_copy(tailT_v, tailT_hbm.at[:, pl.ds(base, SPW)])


@jax.jit
def _sc_pool(candT, scoresT, table):
    mesh = plsc.VectorSubcoreMesh(core_axis_name="c", subcore_axis_name="s")
    f = pl.kernel(
        _sc_pool_body,
        out_type=jax.ShapeDtypeStruct((DIM, N), jnp.float32),
        mesh=mesh,
        compiler_params=pltpu.CompilerParams(needs_layout_passes=False),
        scratch_types=[
            pltpu.VMEM((C, SPW), jnp.int32),
            pltpu.VMEM((C, SPW), jnp.float32),
            pltpu.VMEM((NCHUNK, RPC), jnp.int32),
            pltpu.VMEM((NCHUNK, RPC), jnp.int32),
            pltpu.VMEM((RPC, 2 * DIM), jnp.float32),
            pltpu.VMEM((RPC, 2 * DIM), jnp.float32),
            pltpu.VMEM((DIM, SPW), jnp.float32),
            pltpu.SemaphoreType.DMA,
            pltpu.SemaphoreType.DMA,
        ],
    )
    return f(candT, scoresT, table)


def _relayout_body(embT_ref, p_ref, out_ref):
    p = p_ref[...]
    nstrip = embT_ref.shape[1] // 128
    for k4 in range(nstrip // 4):
        s4 = jnp.concatenate(
            [embT_ref[:, pl.ds((k4 * 4 + a) * 128, 128)] for a in range(4)],
            axis=0)
        mt = jax.lax.dot_general(p, s4, (((0,), (1,)), ((), ())),
                                 preferred_element_type=jnp.float32)
        for a in range(4):
            k = k4 * 4 + a
            out_ref[pl.ds(k * DIM, DIM), :DIM] = mt[:DIM, a * DIM:(a + 1) * DIM]
            out_ref[pl.ds(k * DIM, DIM), DIM:] = mt[DIM:, a * DIM:(a + 1) * DIM]


@jax.jit
def _tc_relayout(embT, p):
    b = 4096
    nblk = -(-embT.shape[1] // (2 * b))
    return pl.pallas_call(
        _relayout_body,
        grid=(nblk,),
        in_specs=[
            pl.BlockSpec((DIM, 2 * b), lambda i: (0, i)),
            pl.BlockSpec((128, 128), lambda i: (0, 0)),
        ],
        out_specs=pl.BlockSpec((b, 2 * DIM), lambda i: (i, 0)),
        out_shape=jax.ShapeDtypeStruct((embT.shape[1] // 2, 2 * DIM),
                                       jnp.float32),
    )(embT, p)


def _assemble_body(span_ref, tail_ref, len_ref, out_ref):
    sv = span_ref[...]
    out_ref[:SPAN, :] = sv.T
    gate = (len_ref[...] > 0).astype(jnp.float32)
    out_ref[SPAN:, :] = tail_ref[...] * gate


@jax.jit
def _assemble(span_flat, tailT, len_row):
    blk = 512
    return pl.pallas_call(
        _assemble_body,
        grid=(N // blk,),
        in_specs=[
            pl.BlockSpec((blk, SPAN), lambda i: (i, 0)),
            pl.BlockSpec((DIM, blk), lambda i: (0, i)),
            pl.BlockSpec((1, blk), lambda i: (0, i)),
        ],
        out_specs=pl.BlockSpec((SPAN + DIM, blk), lambda i: (0, i)),
        out_shape=jax.ShapeDtypeStruct((SPAN + DIM, N), jnp.float32),
    )(span_flat, tailT, len_row)


def kernel(span_vecs, scores, mask_candidates, embed, candidates, len_candidates):
    candT = candidates[0].T
    scoresT = scores[0].T
    j = jnp.arange(128)
    perm = jnp.where(j % 2 == 0, j // 2, DIM + j // 2)
    p = (jnp.arange(128)[None, :] == perm[:, None]).astype(jnp.float32)
    table = _tc_relayout(embed.T, p)
    tailT = _sc_pool(candT, scoresT, table)
    outT = _assemble(span_vecs.reshape(N, SPAN), tailT,
                     len_candidates.reshape(1, N))
    return outT.T.reshape(1, N, SPAN + DIM)


# speedup vs baseline: 5.2833x; 1.1203x over previous
import jax
import jax.numpy as jnp
from jax import lax
from jax.experimental import pallas as pl
from jax.experimental.pallas import tpu as pltpu
from jax.experimental.pallas import tpu_sc as plsc

N = 16384
C = 16
DIM = 64
SPAN = 256
NW = 32
SPW = N // NW
RPC = 128
SPC = RPC // C
NCHUNK = SPW // SPC
L = 16


def _sc_pool_body(candT_hbm, scoresT_hbm, table_hbm, tailT_hbm,
                  candT_v, scoresT_v, pair_v, off_v, rows0_v, rows1_v,
                  tailT_v, sem0, sem1):
    wid = lax.axis_index("s") * 2 + lax.axis_index("c")
    base = wid * SPW

    pltpu.sync_copy(candT_hbm.at[:, pl.ds(base, SPW)], candT_v)
    pltpu.sync_copy(scoresT_hbm.at[:, pl.ds(base, SPW)], scoresT_v)

    iota = lax.iota(jnp.int32, L)

    def build(j, _):
        for jj in range(SPC):
            s = j * SPC + jj
            cv = plsc.load_gather(candT_v, [iota, jnp.full((L,), s, jnp.int32)])
            pair_v[j, pl.ds(jj * C, L)] = lax.shift_right_logical(cv, 1)
            off_v[j, pl.ds(jj * C, L)] = (cv & 1) * DIM
        return ()

    lax.fori_loop(0, NCHUNK, build, (), unroll=False)

    def gather_start(j, rows, s):
        pltpu.async_copy(table_hbm.at[pair_v.at[j]], rows, s)

    def gather_wait(j, rows, s):
        pltpu.make_async_copy(table_hbm.at[pair_v.at[j]], rows, s).wait()

    def compute(j, rows):
        for jj in range(SPC):
            s = j * SPC + jj
            sidx = jnp.full((L,), s, jnp.int32)
            accs = [jnp.zeros((L,), jnp.float32) for _ in range(DIM // L)]
            offv = off_v[j, pl.ds(jj * C, C)]
            for c in range(C):
                r = jj * C + c
                bc = plsc.load_gather(
                    scoresT_v, [jnp.full((L,), c, jnp.int32), sidx])
                off = offv[c]
                for d in range(DIM // L):
                    accs[d] = accs[d] + bc * rows[r, pl.ds(off + d * L, L)]
            for d in range(DIM // L):
                plsc.store_scatter(
                    tailT_v, [d * L + iota, sidx], accs[d])

    gather_start(0, rows0_v, sem0)

    def step(j2, _):
        j = j2 * 2
        gather_wait(j, rows0_v, sem0)
        gather_start(j + 1, rows1_v, sem1)
        compute(j, rows0_v)
        gather_wait(j + 1, rows1_v, sem1)

        @pl.when(j2 + 1 < NCHUNK // 2)
        def _():
            gather_start(j + 2, rows0_v, sem0)

        compute(j + 1, rows1_v)
        return ()

    lax.fori_loop(0, NCHUNK // 2, step, (), unroll=False)

    pltpu.sync_copy(tailT_v, tailT_hbm.at[:, pl.ds(base, SPW)])


@jax.jit
def _sc_pool(candT, scoresT, table):
    mesh = plsc.VectorSubcoreMesh(core_axis_name="c", subcore_axis_name="s")
    f = pl.kernel(
        _sc_pool_body,
        out_type=jax.ShapeDtypeStruct((DIM, N), jnp.float32),
        mesh=mesh,
        compiler_params=pltpu.CompilerParams(needs_layout_passes=False),
        scratch_types=[
            pltpu.VMEM((C, SPW), jnp.int32),
            pltpu.VMEM((C, SPW), jnp.float32),
            pltpu.VMEM((NCHUNK, RPC), jnp.int32),
            pltpu.VMEM((NCHUNK, RPC), jnp.int32),
            pltpu.VMEM((RPC, 2 * DIM), jnp.float32),
            pltpu.VMEM((RPC, 2 * DIM), jnp.float32),
            pltpu.VMEM((DIM, SPW), jnp.float32),
            pltpu.SemaphoreType.DMA,
            pltpu.SemaphoreType.DMA,
        ],
    )
    return f(candT, scoresT, table)


def _relayout_body(embT_ref, p_ref, out_ref):
    p = p_ref[...]
    nstrip = embT_ref.shape[1] // 128
    for k4 in range(nstrip // 4):
        s4 = jnp.concatenate(
            [embT_ref[:, pl.ds((k4 * 4 + a) * 128, 128)] for a in range(4)],
            axis=0)
        mt = jax.lax.dot_general(p, s4, (((0,), (1,)), ((), ())),
                                 preferred_element_type=jnp.float32)
        for a in range(4):
            k = k4 * 4 + a
            out_ref[pl.ds(k * DIM, DIM), :DIM] = mt[:DIM, a * DIM:(a + 1) * DIM]
            out_ref[pl.ds(k * DIM, DIM), DIM:] = mt[DIM:, a * DIM:(a + 1) * DIM]


@jax.jit
def _tc_relayout(embT, p):
    b = 8192
    nblk = -(-embT.shape[1] // (2 * b))
    return pl.pallas_call(
        _relayout_body,
        grid=(nblk,),
        in_specs=[
            pl.BlockSpec((DIM, 2 * b), lambda i: (0, i)),
            pl.BlockSpec((128, 128), lambda i: (0, 0)),
        ],
        out_specs=pl.BlockSpec((b, 2 * DIM), lambda i: (i, 0)),
        out_shape=jax.ShapeDtypeStruct((embT.shape[1] // 2, 2 * DIM),
                                       jnp.float32),
    )(embT, p)


def _assemble_body(span_ref, tail_ref, len_ref, out_ref):
    sv = span_ref[...]
    out_ref[:SPAN, :] = sv.T
    gate = (len_ref[...] > 0).astype(jnp.float32)
    out_ref[SPAN:, :] = tail_ref[...] * gate


@jax.jit
def _assemble(span_flat, tailT, len_row):
    blk = 512
    return pl.pallas_call(
        _assemble_body,
        grid=(N // blk,),
        in_specs=[
            pl.BlockSpec((blk, SPAN), lambda i: (i, 0)),
            pl.BlockSpec((DIM, blk), lambda i: (0, i)),
            pl.BlockSpec((1, blk), lambda i: (0, i)),
        ],
        out_specs=pl.BlockSpec((SPAN + DIM, blk), lambda i: (0, i)),
        out_shape=jax.ShapeDtypeStruct((SPAN + DIM, N), jnp.float32),
    )(span_flat, tailT, len_row)


def kernel(span_vecs, scores, mask_candidates, embed, candidates, len_candidates):
    candT = candidates[0].T
    scoresT = scores[0].T
    j = jnp.arange(128)
    perm = jnp.where(j % 2 == 0, j // 2, DIM + j // 2)
    p = (jnp.arange(128)[None, :] == perm[:, None]).astype(jnp.float32)
    table = _tc_relayout(embed.T, p)
    tailT = _sc_pool(candT, scoresT, table)
    outT = _assemble(span_vecs.reshape(N, SPAN), tailT,
                     len_candidates.reshape(1, N))
    return outT.T.reshape(1, N, SPAN + DIM)


# speedup vs baseline: 5.4565x; 1.0328x over previous
import jax
import jax.numpy as jnp
from jax import lax
from jax.experimental import pallas as pl
from jax.experimental.pallas import tpu as pltpu
from jax.experimental.pallas import tpu_sc as plsc

N = 16384
C = 16
DIM = 64
SPAN = 256
NW = 32
SPW = N // NW
RPC = 128
SPC = RPC // C
NCHUNK = SPW // SPC
L = 16


def _sc_pool_body(candT_hbm, scoresT_hbm, table_hbm, tailT_hbm,
                  candT_v, scoresT_v, pair_v, off_v, rows0_v, rows1_v,
                  tailT_v, sem0, sem1):
    wid = lax.axis_index("s") * 2 + lax.axis_index("c")
    base = wid * SPW

    pltpu.sync_copy(candT_hbm.at[:, pl.ds(base, SPW)], candT_v)
    pltpu.sync_copy(scoresT_hbm.at[:, pl.ds(base, SPW)], scoresT_v)

    iota = lax.iota(jnp.int32, L)

    def build(j, _):
        for jj in range(SPC):
            s = j * SPC + jj
            cv = plsc.load_gather(candT_v, [iota, jnp.full((L,), s, jnp.int32)])
            pair_v[j, pl.ds(jj * C, L)] = lax.shift_right_logical(cv, 1)
            off_v[j, pl.ds(jj * C, L)] = (cv & 1) * DIM
        return ()

    lax.fori_loop(0, NCHUNK, build, (), unroll=False)

    def gather_start(j, rows, s):
        pltpu.async_copy(table_hbm.at[pair_v.at[j]], rows, s)

    def gather_wait(j, rows, s):
        pltpu.make_async_copy(table_hbm.at[pair_v.at[j]], rows, s).wait()

    def compute(j, rows):
        for jj in range(SPC):
            s = j * SPC + jj
            sidx = jnp.full((L,), s, jnp.int32)
            accs = [jnp.zeros((L,), jnp.float32) for _ in range(DIM // L)]
            offv = off_v[j, pl.ds(jj * C, C)]
            for c in range(C):
                r = jj * C + c
                bc = plsc.load_gather(
                    scoresT_v, [jnp.full((L,), c, jnp.int32), sidx])
                off = offv[c]
                for d in range(DIM // L):
                    accs[d] = accs[d] + bc * rows[r, pl.ds(off + d * L, L)]
            for d in range(DIM // L):
                plsc.store_scatter(
                    tailT_v, [d * L + iota, sidx], accs[d])

    gather_start(0, rows0_v, sem0)

    def step(j2, _):
        j = j2 * 2
        gather_wait(j, rows0_v, sem0)
        gather_start(j + 1, rows1_v, sem1)
        compute(j, rows0_v)
        gather_wait(j + 1, rows1_v, sem1)

        @pl.when(j2 + 1 < NCHUNK // 2)
        def _():
            gather_start(j + 2, rows0_v, sem0)

        compute(j + 1, rows1_v)
        return ()

    lax.fori_loop(0, NCHUNK // 2, step, (), unroll=False)

    pltpu.sync_copy(tailT_v, tailT_hbm.at[:, pl.ds(base, SPW)])


@jax.jit
def _sc_pool(candT, scoresT, table):
    mesh = plsc.VectorSubcoreMesh(core_axis_name="c", subcore_axis_name="s")
    f = pl.kernel(
        _sc_pool_body,
        out_type=jax.ShapeDtypeStruct((DIM, N), jnp.float32),
        mesh=mesh,
        compiler_params=pltpu.CompilerParams(needs_layout_passes=False),
        scratch_types=[
            pltpu.VMEM((C, SPW), jnp.int32),
            pltpu.VMEM((C, SPW), jnp.float32),
            pltpu.VMEM((NCHUNK, RPC), jnp.int32),
            pltpu.VMEM((NCHUNK, RPC), jnp.int32),
            pltpu.VMEM((RPC, 2 * DIM), jnp.float32),
            pltpu.VMEM((RPC, 2 * DIM), jnp.float32),
            pltpu.VMEM((DIM, SPW), jnp.float32),
            pltpu.SemaphoreType.DMA,
            pltpu.SemaphoreType.DMA,
        ],
    )
    return f(candT, scoresT, table)


def _relayout_body(embT_ref, p_ref, out_ref):
    p = p_ref[...]
    nstrip = embT_ref.shape[1] // 128
    for k4 in range(nstrip // 4):
        s4 = jnp.concatenate(
            [embT_ref[:, pl.ds((k4 * 4 + a) * 128, 128)] for a in range(4)],
            axis=0)
        mt = jax.lax.dot_general(p, s4, (((0,), (1,)), ((), ())),
                                 preferred_element_type=jnp.float32)
        for a in range(4):
            k = k4 * 4 + a
            out_ref[pl.ds(k * DIM, DIM), :DIM] = mt[:DIM, a * DIM:(a + 1) * DIM]
            out_ref[pl.ds(k * DIM, DIM), DIM:] = mt[DIM:, a * DIM:(a + 1) * DIM]


@jax.jit
def _tc_relayout(embT, p):
    b = 16384
    nblk = -(-embT.shape[1] // (2 * b))
    return pl.pallas_call(
        _relayout_body,
        grid=(nblk,),
        in_specs=[
            pl.BlockSpec((DIM, 2 * b), lambda i: (0, i)),
            pl.BlockSpec((128, 128), lambda i: (0, 0)),
        ],
        out_specs=pl.BlockSpec((b, 2 * DIM), lambda i: (i, 0)),
        out_shape=jax.ShapeDtypeStruct((embT.shape[1] // 2, 2 * DIM),
                                       jnp.float32),
    )(embT, p)


def _assemble_body(span_ref, tail_ref, len_ref, out_ref):
    sv = span_ref[...]
    out_ref[:SPAN, :] = sv.T
    gate = (len_ref[...] > 0).astype(jnp.float32)
    out_ref[SPAN:, :] = tail_ref[...] * gate


@jax.jit
def _assemble(span_flat, tailT, len_row):
    blk = 512
    return pl.pallas_call(
        _assemble_body,
        grid=(N // blk,),
        in_specs=[
            pl.BlockSpec((blk, SPAN), lambda i: (i, 0)),
            pl.BlockSpec((DIM, blk), lambda i: (0, i)),
            pl.BlockSpec((1, blk), lambda i: (0, i)),
        ],
        out_specs=pl.BlockSpec((SPAN + DIM, blk), lambda i: (0, i)),
        out_shape=jax.ShapeDtypeStruct((SPAN + DIM, N), jnp.float32),
    )(span_flat, tailT, len_row)


def kernel(span_vecs, scores, mask_candidates, embed, candidates, len_candidates):
    candT = candidates[0].T
    scoresT = scores[0].T
    j = jnp.arange(128)
    perm = jnp.where(j % 2 == 0, j // 2, DIM + j // 2)
    p = (jnp.arange(128)[None, :] == perm[:, None]).astype(jnp.float32)
    table = _tc_relayout(embed.T, p)
    tailT = _sc_pool(candT, scoresT, table)
    outT = _assemble(span_vecs.reshape(N, SPAN), tailT,
                     len_candidates.reshape(1, N))
    return outT.T.reshape(1, N, SPAN + DIM)
